# baseline (speedup 1.0000x reference)
import functools

import jax
import jax.numpy as jnp
import numpy as np
from jax.experimental import pallas as pl
from jax.experimental.pallas import tpu as pltpu

_TEMP = 0.1
_INV_TEMP = 1.0 / _TEMP
_EXP2C = float(_INV_TEMP / np.log(2.0))
_B = 1024


def _tri_body(i_ref, j_ref, a_r, a_c, pk_i, s_j, ei_j,
              rp, rn, cp, cn, *, b):
    t = pl.program_id(0)
    is_diag = i_ref[t] == j_ref[t]

    raw = jax.lax.dot_general(
        a_r[...], a_c[...], (((1,), (1,)), ((), ())),
        preferred_element_type=jnp.float32)

    s_r = pk_i[:, 0:1]
    r_r = jax.lax.bitcast_convert_type(pk_i[:, 1:2], jnp.int32)
    c_r = jax.lax.bitcast_convert_type(pk_i[:, 2:3], jnp.int32)
    s_c = s_j[...]
    r_c = ei_j[0:1, :]
    c_c = ei_j[1:2, :]

    score_mask = jnp.abs(s_r - s_c) < 0.1
    shared = ((r_r == r_c) | (r_r == c_c)
              | (c_r == r_c) | (c_r == c_c))
    mask = score_mask | shared

    esim = jnp.exp2(raw * _EXP2C)
    pos_t = jnp.where(mask, raw, 0.0)
    neg_t = jnp.where(mask, 0.0, esim)

    rp[...] = jnp.swapaxes(
        jnp.sum(pos_t, axis=1, keepdims=True), 0, 1)[None] * _INV_TEMP
    rn[...] = jnp.swapaxes(jnp.sum(neg_t, axis=1, keepdims=True), 0, 1)[None]
    scale = jnp.where(is_diag, 0.0, 1.0)
    cp[...] = (jnp.sum(pos_t, axis=0, keepdims=True) * (scale * _INV_TEMP))[None]
    cn[...] = (jnp.sum(neg_t, axis=0, keepdims=True) * scale)[None]


def _partials(edge_embeddings, edge_index, structural_scores):
    e, d = edge_embeddings.shape
    b = _B
    nb = e // b
    ei = edge_index.astype(jnp.int32)
    s = structural_scores.astype(jnp.float32)
    emb_bf16 = edge_embeddings.astype(jnp.bfloat16)

    iu, ju = [], []
    fwd = True
    for i in range(nb):
        js = list(range(i, nb))
        if not fwd:
            js.reverse()
        fwd = not fwd
        iu.extend([i] * len(js))
        ju.extend(js)
    tt = len(iu)
    i_arr = jnp.asarray(iu, dtype=jnp.int32)
    j_arr = jnp.asarray(ju, dtype=jnp.int32)

    grid_spec = pltpu.PrefetchScalarGridSpec(
        num_scalar_prefetch=2,
        grid=(tt,),
        in_specs=[
            pl.BlockSpec((b, d), lambda t, i_r, j_r: (i_r[t], 0)),
            pl.BlockSpec((b, d), lambda t, i_r, j_r: (j_r[t], 0)),
            pl.BlockSpec((b, 3), lambda t, i_r, j_r: (i_r[t], 0)),
            pl.BlockSpec((1, b), lambda t, i_r, j_r: (0, j_r[t])),
            pl.BlockSpec((2, b), lambda t, i_r, j_r: (0, j_r[t])),
        ],
        out_specs=[
            pl.BlockSpec((1, 1, b), lambda t, i_r, j_r: (t, 0, 0)),
            pl.BlockSpec((1, 1, b), lambda t, i_r, j_r: (t, 0, 0)),
            pl.BlockSpec((1, 1, b), lambda t, i_r, j_r: (t, 0, 0)),
            pl.BlockSpec((1, 1, b), lambda t, i_r, j_r: (t, 0, 0)),
        ],
    )
    rp, rn, cp, cn = pl.pallas_call(
        functools.partial(_tri_body, b=b),
        grid_spec=grid_spec,
        out_shape=[
            jax.ShapeDtypeStruct((tt, 1, b), jnp.float32),
            jax.ShapeDtypeStruct((tt, 1, b), jnp.float32),
            jax.ShapeDtypeStruct((tt, 1, b), jnp.float32),
            jax.ShapeDtypeStruct((tt, 1, b), jnp.float32),
        ],
        compiler_params=pltpu.CompilerParams(
            dimension_semantics=("arbitrary",),
        ),
        name="contrastive_loss_tri",
    )(
        i_arr, j_arr,
        emb_bf16,
        emb_bf16,
        jnp.concatenate(
            [s.reshape(e, 1),
             jax.lax.bitcast_convert_type(ei[0], jnp.float32).reshape(e, 1),
             jax.lax.bitcast_convert_type(ei[1], jnp.float32).reshape(e, 1)],
            axis=1),
        s.reshape(1, e),
        ei,
    )

    return rp, rn, cp, cn, iu, ju, nb, e


def _row_sums(edge_embeddings, edge_index, structural_scores):
    rp, rn, cp, cn, iu, ju, nb, e = _partials(
        edge_embeddings, edge_index, structural_scores)
    i_arr = jnp.asarray(iu, dtype=jnp.int32)
    j_arr = jnp.asarray(ju, dtype=jnp.int32)
    pos = (jax.ops.segment_sum(rp[:, 0, :], i_arr, num_segments=nb)
           + jax.ops.segment_sum(cp[:, 0, :], j_arr, num_segments=nb)
           ).reshape(e)
    neg = (jax.ops.segment_sum(rn[:, 0, :], i_arr, num_segments=nb)
           + jax.ops.segment_sum(cn[:, 0, :], j_arr, num_segments=nb)
           ).reshape(e)
    return pos, neg


def _finalize_body(rp, rn, cp, cn, out, *, groups_i, groups_j, e):
    nb = len(groups_i)
    pos_rows = []
    neg_rows = []
    for blk in range(nb):
        p = jnp.zeros((1, rp.shape[1]), jnp.float32)
        n = jnp.zeros((1, rn.shape[1]), jnp.float32)
        for t in groups_i[blk]:
            p = p + rp[t:t + 1, :]
            n = n + rn[t:t + 1, :]
        for t in groups_j[blk]:
            p = p + cp[t:t + 1, :]
            n = n + cn[t:t + 1, :]
        pos_rows.append(p)
        neg_rows.append(n)
    pos = jnp.concatenate(pos_rows, axis=0)
    neg = jnp.concatenate(neg_rows, axis=0)
    loss = -jnp.log(pos / (pos + neg + 1e-8))
    out[0, 0] = jnp.sum(loss) / e


@jax.jit
def kernel(edge_embeddings, edge_index, structural_scores):
    rp, rn, cp, cn, iu, ju, nb, e = _partials(
        edge_embeddings, edge_index, structural_scores)
    iu = [int(x) for x in iu]
    ju = [int(x) for x in ju]
    groups_i = [[t for t, i in enumerate(iu) if i == blk] for blk in range(nb)]
    groups_j = [[t for t, (i, j) in enumerate(zip(iu, ju))
                 if j == blk and i != j] for blk in range(nb)]
    out = pl.pallas_call(
        functools.partial(_finalize_body, groups_i=groups_i,
                          groups_j=groups_j, e=e),
        out_specs=pl.BlockSpec((1, 1), memory_space=pltpu.SMEM),
        out_shape=jax.ShapeDtypeStruct((1, 1), jnp.float32),
        name="contrastive_loss_finalize",
    )(rp.reshape(rp.shape[0], -1), rn.reshape(rn.shape[0], -1),
      cp.reshape(cp.shape[0], -1), cn.reshape(cn.shape[0], -1))
    return out[0, 0]


# speedup vs baseline: 1.0484x; 1.0484x over previous
import functools

import jax
import jax.numpy as jnp
import numpy as np
from jax.experimental import pallas as pl
from jax.experimental.pallas import tpu as pltpu

_TEMP = 0.1
_INV_TEMP = 1.0 / _TEMP
_EXP2C = float(_INV_TEMP / np.log(2.0))
_B = 1024


def _tri_body(i_ref, j_ref, a_r, a_c, pk_i, s_j, ei_j,
              rp, rn, cp, cn, *, b):
    t = pl.program_id(0)
    is_diag = i_ref[t] == j_ref[t]

    raw = jax.lax.dot_general(
        a_r[...], a_c[...], (((1,), (1,)), ((), ())),
        preferred_element_type=jnp.float32)

    s_r = pk_i[:, 0:1]
    r_r = jax.lax.bitcast_convert_type(pk_i[:, 1:2], jnp.int32)
    c_r = jax.lax.bitcast_convert_type(pk_i[:, 2:3], jnp.int32)
    s_c = s_j[...]
    r_c = ei_j[0:1, :]
    c_c = ei_j[1:2, :]

    score_mask = jnp.abs(s_r - s_c) < 0.1
    shared = ((r_r == r_c) | (r_r == c_c)
              | (c_r == r_c) | (c_r == c_c))
    mask = score_mask | shared

    esim = jnp.exp2(raw * _EXP2C)
    pos_t = jnp.where(mask, raw, 0.0)
    neg_t = jnp.where(mask, 0.0, esim)

    rp[...] = jnp.swapaxes(
        jnp.sum(pos_t, axis=1, keepdims=True), 0, 1)[None] * _INV_TEMP
    rn[...] = jnp.swapaxes(jnp.sum(neg_t, axis=1, keepdims=True), 0, 1)[None]
    scale = jnp.where(is_diag, 0.0, 1.0)
    cp[...] = (jnp.sum(pos_t, axis=0, keepdims=True) * (scale * _INV_TEMP))[None]
    cn[...] = (jnp.sum(neg_t, axis=0, keepdims=True) * scale)[None]


def _partials(edge_embeddings, edge_index, structural_scores):
    e, d = edge_embeddings.shape
    b = _B
    nb = e // b
    ei = edge_index.astype(jnp.int32)
    s = structural_scores.astype(jnp.float32)

    iu, ju = [], []
    fwd = True
    for i in range(nb):
        js = list(range(i, nb))
        if not fwd:
            js.reverse()
        fwd = not fwd
        iu.extend([i] * len(js))
        ju.extend(js)
    tt = len(iu)
    i_arr = jnp.asarray(iu, dtype=jnp.int32)
    j_arr = jnp.asarray(ju, dtype=jnp.int32)

    grid_spec = pltpu.PrefetchScalarGridSpec(
        num_scalar_prefetch=2,
        grid=(tt,),
        in_specs=[
            pl.BlockSpec((b, d), lambda t, i_r, j_r: (i_r[t], 0)),
            pl.BlockSpec((b, d), lambda t, i_r, j_r: (j_r[t], 0)),
            pl.BlockSpec((b, 3), lambda t, i_r, j_r: (i_r[t], 0)),
            pl.BlockSpec((1, b), lambda t, i_r, j_r: (0, j_r[t])),
            pl.BlockSpec((2, b), lambda t, i_r, j_r: (0, j_r[t])),
        ],
        out_specs=[
            pl.BlockSpec((1, 1, b), lambda t, i_r, j_r: (t, 0, 0)),
            pl.BlockSpec((1, 1, b), lambda t, i_r, j_r: (t, 0, 0)),
            pl.BlockSpec((1, 1, b), lambda t, i_r, j_r: (t, 0, 0)),
            pl.BlockSpec((1, 1, b), lambda t, i_r, j_r: (t, 0, 0)),
        ],
    )
    rp, rn, cp, cn = pl.pallas_call(
        functools.partial(_tri_body, b=b),
        grid_spec=grid_spec,
        out_shape=[
            jax.ShapeDtypeStruct((tt, 1, b), jnp.float32),
            jax.ShapeDtypeStruct((tt, 1, b), jnp.float32),
            jax.ShapeDtypeStruct((tt, 1, b), jnp.float32),
            jax.ShapeDtypeStruct((tt, 1, b), jnp.float32),
        ],
        compiler_params=pltpu.CompilerParams(
            dimension_semantics=("arbitrary",),
        ),
        name="contrastive_loss_tri",
    )(
        i_arr, j_arr,
        edge_embeddings,
        edge_embeddings,
        jnp.concatenate(
            [s.reshape(e, 1),
             jax.lax.bitcast_convert_type(ei[0], jnp.float32).reshape(e, 1),
             jax.lax.bitcast_convert_type(ei[1], jnp.float32).reshape(e, 1)],
            axis=1),
        s.reshape(1, e),
        ei,
    )

    return rp, rn, cp, cn, iu, ju, nb, e


def _row_sums(edge_embeddings, edge_index, structural_scores):
    rp, rn, cp, cn, iu, ju, nb, e = _partials(
        edge_embeddings, edge_index, structural_scores)
    i_arr = jnp.asarray(iu, dtype=jnp.int32)
    j_arr = jnp.asarray(ju, dtype=jnp.int32)
    pos = (jax.ops.segment_sum(rp[:, 0, :], i_arr, num_segments=nb)
           + jax.ops.segment_sum(cp[:, 0, :], j_arr, num_segments=nb)
           ).reshape(e)
    neg = (jax.ops.segment_sum(rn[:, 0, :], i_arr, num_segments=nb)
           + jax.ops.segment_sum(cn[:, 0, :], j_arr, num_segments=nb)
           ).reshape(e)
    return pos, neg


def _finalize_body(rp, rn, cp, cn, out, *, groups_i, groups_j, e):
    nb = len(groups_i)
    pos_rows = []
    neg_rows = []
    for blk in range(nb):
        p = jnp.zeros((1, rp.shape[1]), jnp.float32)
        n = jnp.zeros((1, rn.shape[1]), jnp.float32)
        for t in groups_i[blk]:
            p = p + rp[t:t + 1, :]
            n = n + rn[t:t + 1, :]
        for t in groups_j[blk]:
            p = p + cp[t:t + 1, :]
            n = n + cn[t:t + 1, :]
        pos_rows.append(p)
        neg_rows.append(n)
    pos = jnp.concatenate(pos_rows, axis=0)
    neg = jnp.concatenate(neg_rows, axis=0)
    loss = -jnp.log(pos / (pos + neg + 1e-8))
    out[0, 0] = jnp.sum(loss) / e


@jax.jit
def kernel(edge_embeddings, edge_index, structural_scores):
    rp, rn, cp, cn, iu, ju, nb, e = _partials(
        edge_embeddings, edge_index, structural_scores)
    iu = [int(x) for x in iu]
    ju = [int(x) for x in ju]
    groups_i = [[t for t, i in enumerate(iu) if i == blk] for blk in range(nb)]
    groups_j = [[t for t, (i, j) in enumerate(zip(iu, ju))
                 if j == blk and i != j] for blk in range(nb)]
    out = pl.pallas_call(
        functools.partial(_finalize_body, groups_i=groups_i,
                          groups_j=groups_j, e=e),
        out_specs=pl.BlockSpec((1, 1), memory_space=pltpu.SMEM),
        out_shape=jax.ShapeDtypeStruct((1, 1), jnp.float32),
        name="contrastive_loss_finalize",
    )(rp.reshape(rp.shape[0], -1), rn.reshape(rn.shape[0], -1),
      cp.reshape(cp.shape[0], -1), cn.reshape(cn.shape[0], -1))
    return out[0, 0]


# speedup vs baseline: 1.0686x; 1.0192x over previous
import functools

import jax
import jax.numpy as jnp
import numpy as np
from jax.experimental import pallas as pl
from jax.experimental.pallas import tpu as pltpu

_TEMP = 0.1
_B = 1024


def _tri_body(i_ref, j_ref, a_r, a_c, pk_i, s_j, ei_j,
              rp, rn, cp, cn, *, b):
    t = pl.program_id(0)
    is_diag = i_ref[t] == j_ref[t]

    sim = jax.lax.dot_general(
        a_r[...], a_c[...], (((1,), (1,)), ((), ())),
        preferred_element_type=jnp.float32) / _TEMP

    s_r = pk_i[:, 0:1]
    r_r = jax.lax.bitcast_convert_type(pk_i[:, 1:2], jnp.int32)
    c_r = jax.lax.bitcast_convert_type(pk_i[:, 2:3], jnp.int32)
    s_c = s_j[...]
    r_c = ei_j[0:1, :]
    c_c = ei_j[1:2, :]

    score_mask = jnp.abs(s_r - s_c) < 0.1
    shared = ((r_r == r_c) | (r_r == c_c)
              | (c_r == r_c) | (c_r == c_c))
    mask = score_mask | shared

    esim = jnp.exp(sim)
    pos_t = jnp.where(mask, sim, 0.0)
    neg_t = jnp.where(mask, 0.0, esim)

    rp[...] = jnp.swapaxes(jnp.sum(pos_t, axis=1, keepdims=True), 0, 1)[None]
    rn[...] = jnp.swapaxes(jnp.sum(neg_t, axis=1, keepdims=True), 0, 1)[None]
    scale = jnp.where(is_diag, 0.0, 1.0)
    cp[...] = (jnp.sum(pos_t, axis=0, keepdims=True) * scale)[None]
    cn[...] = (jnp.sum(neg_t, axis=0, keepdims=True) * scale)[None]


def _partials(edge_embeddings, edge_index, structural_scores):
    e, d = edge_embeddings.shape
    b = _B
    nb = e // b
    ei = edge_index.astype(jnp.int32)
    s = structural_scores.astype(jnp.float32)

    iu, ju = [], []
    fwd = True
    for i in range(nb):
        js = list(range(i, nb))
        if not fwd:
            js.reverse()
        fwd = not fwd
        iu.extend([i] * len(js))
        ju.extend(js)
    tt = len(iu)
    i_arr = jnp.asarray(iu, dtype=jnp.int32)
    j_arr = jnp.asarray(ju, dtype=jnp.int32)

    grid_spec = pltpu.PrefetchScalarGridSpec(
        num_scalar_prefetch=2,
        grid=(tt,),
        in_specs=[
            pl.BlockSpec((b, d), lambda t, i_r, j_r: (i_r[t], 0)),
            pl.BlockSpec((b, d), lambda t, i_r, j_r: (j_r[t], 0)),
            pl.BlockSpec((b, 3), lambda t, i_r, j_r: (i_r[t], 0)),
            pl.BlockSpec((1, b), lambda t, i_r, j_r: (0, j_r[t])),
            pl.BlockSpec((2, b), lambda t, i_r, j_r: (0, j_r[t])),
        ],
        out_specs=[
            pl.BlockSpec((1, 1, b), lambda t, i_r, j_r: (t, 0, 0)),
            pl.BlockSpec((1, 1, b), lambda t, i_r, j_r: (t, 0, 0)),
            pl.BlockSpec((1, 1, b), lambda t, i_r, j_r: (t, 0, 0)),
            pl.BlockSpec((1, 1, b), lambda t, i_r, j_r: (t, 0, 0)),
        ],
    )
    rp, rn, cp, cn = pl.pallas_call(
        functools.partial(_tri_body, b=b),
        grid_spec=grid_spec,
        out_shape=[
            jax.ShapeDtypeStruct((tt, 1, b), jnp.float32),
            jax.ShapeDtypeStruct((tt, 1, b), jnp.float32),
            jax.ShapeDtypeStruct((tt, 1, b), jnp.float32),
            jax.ShapeDtypeStruct((tt, 1, b), jnp.float32),
        ],
        compiler_params=pltpu.CompilerParams(
            dimension_semantics=("arbitrary",),
        ),
        name="contrastive_loss_tri",
    )(
        i_arr, j_arr,
        edge_embeddings,
        edge_embeddings,
        jnp.concatenate(
            [s.reshape(e, 1),
             jax.lax.bitcast_convert_type(ei[0], jnp.float32).reshape(e, 1),
             jax.lax.bitcast_convert_type(ei[1], jnp.float32).reshape(e, 1)],
            axis=1),
        s.reshape(1, e),
        ei,
    )

    return rp, rn, cp, cn, iu, ju, nb, e


def _row_sums(edge_embeddings, edge_index, structural_scores):
    rp, rn, cp, cn, iu, ju, nb, e = _partials(
        edge_embeddings, edge_index, structural_scores)
    i_arr = jnp.asarray(iu, dtype=jnp.int32)
    j_arr = jnp.asarray(ju, dtype=jnp.int32)
    pos = (jax.ops.segment_sum(rp[:, 0, :], i_arr, num_segments=nb)
           + jax.ops.segment_sum(cp[:, 0, :], j_arr, num_segments=nb)
           ).reshape(e)
    neg = (jax.ops.segment_sum(rn[:, 0, :], i_arr, num_segments=nb)
           + jax.ops.segment_sum(cn[:, 0, :], j_arr, num_segments=nb)
           ).reshape(e)
    return pos, neg


def _finalize_body(rp, rn, cp, cn, out, *, groups_i, groups_j, e):
    nb = len(groups_i)
    pos_rows = []
    neg_rows = []
    for blk in range(nb):
        p = jnp.zeros((1, rp.shape[1]), jnp.float32)
        n = jnp.zeros((1, rn.shape[1]), jnp.float32)
        for t in groups_i[blk]:
            p = p + rp[t:t + 1, :]
            n = n + rn[t:t + 1, :]
        for t in groups_j[blk]:
            p = p + cp[t:t + 1, :]
            n = n + cn[t:t + 1, :]
        pos_rows.append(p)
        neg_rows.append(n)
    pos = jnp.concatenate(pos_rows, axis=0)
    neg = jnp.concatenate(neg_rows, axis=0)
    loss = -jnp.log(pos / (pos + neg + 1e-8))
    out[0, 0] = jnp.sum(loss) / e


@jax.jit
def kernel(edge_embeddings, edge_index, structural_scores):
    rp, rn, cp, cn, iu, ju, nb, e = _partials(
        edge_embeddings, edge_index, structural_scores)
    iu = [int(x) for x in iu]
    ju = [int(x) for x in ju]
    groups_i = [[t for t, i in enumerate(iu) if i == blk] for blk in range(nb)]
    groups_j = [[t for t, (i, j) in enumerate(zip(iu, ju))
                 if j == blk and i != j] for blk in range(nb)]
    out = pl.pallas_call(
        functools.partial(_finalize_body, groups_i=groups_i,
                          groups_j=groups_j, e=e),
        out_specs=pl.BlockSpec((1, 1), memory_space=pltpu.SMEM),
        out_shape=jax.ShapeDtypeStruct((1, 1), jnp.float32),
        name="contrastive_loss_finalize",
    )(rp.reshape(rp.shape[0], -1), rn.reshape(rn.shape[0], -1),
      cp.reshape(cp.shape[0], -1), cn.reshape(cn.shape[0], -1))
    return out[0, 0]


# speedup vs baseline: 1.0776x; 1.0085x over previous
import functools

import jax
import jax.numpy as jnp
import numpy as np
from jax.experimental import pallas as pl
from jax.experimental.pallas import tpu as pltpu

_TEMP = 0.1
_B = 1024


def _tri_body(i_ref, j_ref, a_r, a_c, pk_i, s_j, ei_j, out, *, b):
    t = pl.program_id(0)
    is_diag = i_ref[t] == j_ref[t]

    sim = jax.lax.dot_general(
        a_r[...], a_c[...], (((1,), (1,)), ((), ())),
        preferred_element_type=jnp.float32) / _TEMP

    s_r = pk_i[:, 0:1]
    r_r = jax.lax.bitcast_convert_type(pk_i[:, 1:2], jnp.int32)
    c_r = jax.lax.bitcast_convert_type(pk_i[:, 2:3], jnp.int32)
    s_c = s_j[...]
    r_c = ei_j[0:1, :]
    c_c = ei_j[1:2, :]

    score_mask = jnp.abs(s_r - s_c) < 0.1
    shared = ((r_r == r_c) | (r_r == c_c)
              | (c_r == r_c) | (c_r == c_c))
    mask = score_mask | shared

    esim = jnp.exp(sim)
    pos_t = jnp.where(mask, sim, 0.0)
    neg_t = jnp.where(mask, 0.0, esim)

    rpv = jnp.swapaxes(jnp.sum(pos_t, axis=1, keepdims=True), 0, 1)
    rnv = jnp.swapaxes(jnp.sum(neg_t, axis=1, keepdims=True), 0, 1)
    scale = jnp.where(is_diag, 0.0, 1.0)
    cpv = jnp.sum(pos_t, axis=0, keepdims=True) * scale
    cnv = jnp.sum(neg_t, axis=0, keepdims=True) * scale
    out[...] = jnp.concatenate([rpv, rnv, cpv, cnv], axis=0)[None]


def _partials(edge_embeddings, edge_index, structural_scores):
    e, d = edge_embeddings.shape
    b = _B
    nb = e // b
    ei = edge_index.astype(jnp.int32)
    s = structural_scores.astype(jnp.float32)

    iu, ju = [], []
    fwd = True
    for i in range(nb):
        js = list(range(i, nb))
        if not fwd:
            js.reverse()
        fwd = not fwd
        iu.extend([i] * len(js))
        ju.extend(js)
    tt = len(iu)
    i_arr = jnp.asarray(iu, dtype=jnp.int32)
    j_arr = jnp.asarray(ju, dtype=jnp.int32)

    grid_spec = pltpu.PrefetchScalarGridSpec(
        num_scalar_prefetch=2,
        grid=(tt,),
        in_specs=[
            pl.BlockSpec((b, d), lambda t, i_r, j_r: (i_r[t], 0)),
            pl.BlockSpec((b, d), lambda t, i_r, j_r: (j_r[t], 0)),
            pl.BlockSpec((b, 3), lambda t, i_r, j_r: (i_r[t], 0)),
            pl.BlockSpec((1, b), lambda t, i_r, j_r: (0, j_r[t])),
            pl.BlockSpec((2, b), lambda t, i_r, j_r: (0, j_r[t])),
        ],
        out_specs=pl.BlockSpec((1, 4, b), lambda t, i_r, j_r: (t, 0, 0)),
    )
    packed = pl.pallas_call(
        functools.partial(_tri_body, b=b),
        grid_spec=grid_spec,
        out_shape=jax.ShapeDtypeStruct((tt, 4, b), jnp.float32),
        compiler_params=pltpu.CompilerParams(
            dimension_semantics=("arbitrary",),
        ),
        name="contrastive_loss_tri",
    )(
        i_arr, j_arr,
        edge_embeddings,
        edge_embeddings,
        jnp.concatenate(
            [s.reshape(e, 1),
             jax.lax.bitcast_convert_type(ei[0], jnp.float32).reshape(e, 1),
             jax.lax.bitcast_convert_type(ei[1], jnp.float32).reshape(e, 1)],
            axis=1),
        s.reshape(1, e),
        ei,
    )

    return packed, iu, ju, nb, e


def _row_sums(edge_embeddings, edge_index, structural_scores):
    packed, iu, ju, nb, e = _partials(
        edge_embeddings, edge_index, structural_scores)
    i_arr = jnp.asarray(iu, dtype=jnp.int32)
    j_arr = jnp.asarray(ju, dtype=jnp.int32)
    pos = (jax.ops.segment_sum(packed[:, 0, :], i_arr, num_segments=nb)
           + jax.ops.segment_sum(packed[:, 2, :], j_arr, num_segments=nb)
           ).reshape(e)
    neg = (jax.ops.segment_sum(packed[:, 1, :], i_arr, num_segments=nb)
           + jax.ops.segment_sum(packed[:, 3, :], j_arr, num_segments=nb)
           ).reshape(e)
    return pos, neg


def _finalize_body(pk, out, *, groups_i, groups_j, e):
    nb = len(groups_i)
    pos_rows = []
    neg_rows = []
    for blk in range(nb):
        p = jnp.zeros((1, pk.shape[1]), jnp.float32)
        n = jnp.zeros((1, pk.shape[1]), jnp.float32)
        for t in groups_i[blk]:
            p = p + pk[4 * t:4 * t + 1, :]
            n = n + pk[4 * t + 1:4 * t + 2, :]
        for t in groups_j[blk]:
            p = p + pk[4 * t + 2:4 * t + 3, :]
            n = n + pk[4 * t + 3:4 * t + 4, :]
        pos_rows.append(p)
        neg_rows.append(n)
    pos = jnp.concatenate(pos_rows, axis=0)
    neg = jnp.concatenate(neg_rows, axis=0)
    loss = -jnp.log(pos / (pos + neg + 1e-8))
    out[0, 0] = jnp.sum(loss) / e


@jax.jit
def kernel(edge_embeddings, edge_index, structural_scores):
    packed, iu, ju, nb, e = _partials(
        edge_embeddings, edge_index, structural_scores)
    iu = [int(x) for x in iu]
    ju = [int(x) for x in ju]
    groups_i = [[t for t, i in enumerate(iu) if i == blk] for blk in range(nb)]
    groups_j = [[t for t, (i, j) in enumerate(zip(iu, ju))
                 if j == blk and i != j] for blk in range(nb)]
    out = pl.pallas_call(
        functools.partial(_finalize_body, groups_i=groups_i,
                          groups_j=groups_j, e=e),
        out_specs=pl.BlockSpec((1, 1), memory_space=pltpu.SMEM),
        out_shape=jax.ShapeDtypeStruct((1, 1), jnp.float32),
        name="contrastive_loss_finalize",
    )(packed.reshape(packed.shape[0] * 4, -1))
    return out[0, 0]
